# baseline (speedup 1.0000x reference)
import functools

import jax
import jax.numpy as jnp
import numpy as np
from jax import lax
from jax.experimental import pallas as pl
from jax.experimental.pallas import tpu as pltpu
from jax.experimental.pallas import tpu_sc as plsc

_NC, _NS, _L = 2, 16, 16
_NW = _NC * _NS
_MESH = dict(core_axis_name="c", subcore_axis_name="s",
             num_cores=_NC, num_subcores=_NS)
_SC_PARAMS = pltpu.CompilerParams(use_tc_tiling_on_sc=False,
                                  needs_layout_passes=False)
_CLAMP = 60.0
_DSPREAD = 4096


def _npad(n, spread=0):
    return 512 * ((n + 1 + spread + 511) // 512)


def _ch(w):
    return 80 if w <= 64 else 40


def _mm_body(a_ref, b_ref, bias_ref, o_ref):
    o_ref[...] = (
        jnp.dot(a_ref[...], b_ref[...], preferred_element_type=jnp.float32)
        + bias_ref[...]
    )


def _mm(a, b, bias=None):
    m, k = a.shape
    n = b.shape[1]
    if bias is None:
        bias = jnp.zeros((1, n), jnp.float32)
    else:
        bias = bias.reshape(1, n)
    block_m = m if m * n * 4 <= 4 * 1024 * 1024 else 8000
    return pl.pallas_call(
        _mm_body,
        grid=(m // block_m,),
        in_specs=[
            pl.BlockSpec((block_m, k), lambda i: (i, 0)),
            pl.BlockSpec((k, n), lambda i: (0, 0)),
            pl.BlockSpec((1, n), lambda i: (0, 0)),
        ],
        out_specs=pl.BlockSpec((block_m, n), lambda i: (i, 0)),
        out_shape=jax.ShapeDtypeStruct((m, n), jnp.float32),
    )(a, b, bias)


def _combine_mlp_body(p_ref, xd_ref, w1_ref, b1_ref, w2_ref, b2_ref, o_ref):
    w = xd_ref.shape[1]
    den = p_ref[0, :, :w] + p_ref[1, :, :w]
    num = p_ref[0, :, w:] + p_ref[1, :, w:]
    out = jnp.where(den > 0.0, num / jnp.where(den > 0.0, den, 1.0), 0.0)
    out = out + xd_ref[...]
    h1 = jax.nn.relu(
        jnp.dot(out, w1_ref[...], preferred_element_type=jnp.float32) + b1_ref[...]
    )
    o_ref[...] = (
        jnp.dot(h1, w2_ref[...], preferred_element_type=jnp.float32) + b2_ref[...]
    )


def _combine_mlp(parts, xd, w1, b1, w2, b2):
    n, w = xd.shape
    bn = 2000 if n >= 2000 else n
    return pl.pallas_call(
        _combine_mlp_body,
        grid=(n // bn,),
        in_specs=[
            pl.BlockSpec((2, bn, 2 * w), lambda i: (0, i, 0)),
            pl.BlockSpec((bn, w), lambda i: (i, 0)),
            pl.BlockSpec(w1.shape, lambda i: (0, 0)),
            pl.BlockSpec((1, 2 * w), lambda i: (0, 0)),
            pl.BlockSpec(w2.shape, lambda i: (0, 0)),
            pl.BlockSpec((1, w), lambda i: (0, 0)),
        ],
        out_specs=pl.BlockSpec((bn, w), lambda i: (i, 0)),
        out_shape=jax.ShapeDtypeStruct((n, w), jnp.float32),
    )(parts[:, :n, :], xd, w1, b1.reshape(1, -1), w2, b2.reshape(1, -1))


def _scorer_body(pa_ref, h_ref, wrel_ref, brel_ref, wroot_ref, o_ref):
    agg = pa_ref[0] + pa_ref[1]
    s = (
        jnp.dot(agg, wrel_ref[...], preferred_element_type=jnp.float32)
        + jnp.dot(h_ref[...], wroot_ref[...], preferred_element_type=jnp.float32)
        + brel_ref[...]
    )
    o_ref[...] = jnp.tanh(s)


def _scorer(parts, h, wrel, brel, wroot):
    n, w = h.shape
    return pl.pallas_call(
        _scorer_body,
        out_shape=jax.ShapeDtypeStruct((n, 1), jnp.float32),
    )(parts[:, :n, :], h, wrel, brel.reshape(1, 1), wroot)


def _scale_relu_body(x_ref, v_ref, o_ref):
    o_ref[...] = jax.nn.relu(x_ref[...] * v_ref[...])


def _scale_relu(x, vals):
    k, w = x.shape
    return pl.pallas_call(
        _scale_relu_body,
        out_shape=jax.ShapeDtypeStruct((k, w), jnp.float32),
    )(x, vals.reshape(k, 1))


def _head_body(h_ref, w1_ref, b1_ref, w2_ref, b2_ref, cnt_ref, o_ref):
    s = jnp.sum(h_ref[...], axis=0, keepdims=True)
    h = s / jnp.maximum(cnt_ref[0, 0], 1.0)
    h = jnp.dot(h, w1_ref[...], preferred_element_type=jnp.float32) + b1_ref[...]
    h = jnp.dot(h, w2_ref[...], preferred_element_type=jnp.float32) + b2_ref[...]
    o_ref[...] = h - jax.scipy.special.logsumexp(h, axis=-1, keepdims=True)


def _head(h, p, cnt):
    return pl.pallas_call(
        _head_body,
        out_shape=jax.ShapeDtypeStruct((1, 10), jnp.float32),
    )(h, p["dense1"]["W"], p["dense1"]["b"][None, :],
      p["dense2"]["W"], p["dense2"]["b"][None, :],
      jnp.full((1, 1), cnt, jnp.float32))


def _zero_acc(acc, zbuf, s, rows_s, width):
    zb = zbuf.shape[0]
    for r in range(zb):
        for q in range(width // _L):
            zbuf[r, pl.ds(q * _L, _L)] = jnp.zeros((_L,), jnp.float32)

    def zrow(i, _):
        pltpu.sync_copy(zbuf, acc.at[pl.ds(s * rows_s + i * zb, zb)])
        return 0

    lax.fori_loop(0, rows_s // zb, zrow, 0)


def _sc_edge_pass(tab, e3, src2, seg2, npad):
    nch, ch, w = e3.shape
    nch_w = nch // _NW
    rows_s = npad // _NS

    @functools.partial(
        pl.kernel,
        out_type=jax.ShapeDtypeStruct((_NC, npad, 2 * w), jnp.float32),
        mesh=plsc.VectorSubcoreMesh(**_MESH),
        scratch_types=[
            pltpu.VMEM((nch_w, ch), jnp.int32),
            pltpu.VMEM((nch_w, ch), jnp.int32),
            pltpu.VMEM((ch, w), jnp.float32),
            pltpu.VMEM((ch, w), jnp.float32),
            pltpu.VMEM((ch, 2 * w), jnp.float32),
            pltpu.VMEM((8, 2 * w), jnp.float32),
            pltpu.VMEM_SHARED((npad, 2 * w), jnp.float32),
            pltpu.SemaphoreType.DMA,
        ],
        compiler_params=_SC_PARAMS,
    )
    def k(tab_hbm, e_hbm, src_hbm, seg_hbm, out_hbm,
          srcv, segv, xsb, eb, stage, zbuf, acc, sem):
        c = lax.axis_index("c")
        s = lax.axis_index("s")
        wid = s * _NC + c
        _zero_acc(acc, zbuf, s, rows_s, 2 * w)
        pltpu.sync_copy(src_hbm.at[pl.ds(wid * nch_w, nch_w)], srcv)
        pltpu.sync_copy(seg_hbm.at[pl.ds(wid * nch_w, nch_w)], segv)
        plsc.subcore_barrier()

        def chunk(i, _):
            cp = pltpu.async_copy(tab_hbm.at[srcv.at[i]], xsb, sem)
            pltpu.sync_copy(e_hbm.at[wid * nch_w + i], eb)
            cp.wait()
            for r in range(ch):
                for q in range(w // _L):
                    sl = pl.ds(q * _L, _L)
                    m = jnp.maximum(xsb[r, sl] + eb[r, sl], 0.0) + 1e-7
                    ex = jnp.exp(m - _CLAMP)
                    stage[r, sl] = ex
                    stage[r, pl.ds(w + q * _L, _L)] = ex * m
            pltpu.sync_copy(stage, acc.at[segv.at[i]], add=True)
            return 0

        lax.fori_loop(0, nch_w, chunk, 0)
        plsc.subcore_barrier()
        pltpu.sync_copy(acc.at[pl.ds(s * rows_s, rows_s)],
                        out_hbm.at[c, pl.ds(s * rows_s, rows_s)])

    return k(tab, e3, src2, seg2)


def _sc_agg_pass(tab, src2, seg2, npad):
    nch, ch = src2.shape
    w = tab.shape[1]
    nch_w = nch // _NW
    rows_s = npad // _NS

    @functools.partial(
        pl.kernel,
        out_type=jax.ShapeDtypeStruct((_NC, npad, w), jnp.float32),
        mesh=plsc.VectorSubcoreMesh(**_MESH),
        scratch_types=[
            pltpu.VMEM((nch_w, ch), jnp.int32),
            pltpu.VMEM((nch_w, ch), jnp.int32),
            pltpu.VMEM((ch, w), jnp.float32),
            pltpu.VMEM((8, w), jnp.float32),
            pltpu.VMEM_SHARED((npad, w), jnp.float32),
            pltpu.SemaphoreType.DMA,
        ],
        compiler_params=_SC_PARAMS,
    )
    def k(tab_hbm, src_hbm, seg_hbm, out_hbm, srcv, segv, buf, zbuf, acc, sem):
        c = lax.axis_index("c")
        s = lax.axis_index("s")
        wid = s * _NC + c
        _zero_acc(acc, zbuf, s, rows_s, w)
        pltpu.sync_copy(src_hbm.at[pl.ds(wid * nch_w, nch_w)], srcv)
        pltpu.sync_copy(seg_hbm.at[pl.ds(wid * nch_w, nch_w)], segv)
        plsc.subcore_barrier()

        def chunk(i, _):
            pltpu.async_copy(tab_hbm.at[srcv.at[i]], buf, sem).wait()
            pltpu.sync_copy(buf, acc.at[segv.at[i]], add=True)
            return 0

        lax.fori_loop(0, nch_w, chunk, 0)
        plsc.subcore_barrier()
        pltpu.sync_copy(acc.at[pl.ds(s * rows_s, rows_s)],
                        out_hbm.at[c, pl.ds(s * rows_s, rows_s)])

    return k(tab, src2, seg2)


def _sc_topk(score_pad, k, n):
    ntop = score_pad.shape[0]
    nv = ntop // _L

    @functools.partial(
        pl.kernel,
        out_type=(jax.ShapeDtypeStruct((k,), jnp.int32),
                  jax.ShapeDtypeStruct((k,), jnp.float32)),
        mesh=plsc.VectorSubcoreMesh(**_MESH),
        scratch_types=[
            pltpu.VMEM((ntop,), jnp.float32),
            pltpu.VMEM((ntop,), jnp.uint32),
            pltpu.VMEM((k + _L,), jnp.int32),
            pltpu.VMEM((k + _L,), jnp.float32),
            pltpu.SemaphoreType.DMA,
        ],
        compiler_params=_SC_PARAMS,
    )
    def kk(sc_hbm, perm_hbm, vals_hbm, sv, kv, pidx, pval, sem):
        c = lax.axis_index("c")
        s = lax.axis_index("s")
        wid = s * _NC + c

        @pl.when(wid == 0)
        def _():
            pltpu.sync_copy(sc_hbm, sv)

            def mkkey(j, _):
                sl = pl.ds(j * _L, _L)
                u = plsc.bitcast(sv[sl], jnp.uint32)
                sr = lax.shift_right_logical(u, jnp.uint32(31))
                msk = sr * jnp.uint32(0x7FFFFFFF) + jnp.uint32(0x80000000)
                kv[sl] = u ^ msk
                return 0

            lax.fori_loop(0, nv, mkkey, 0)

            def count_ge(thr):
                def cbody(j, acc):
                    return acc + jnp.where(
                        kv[pl.ds(j * _L, _L)] >= thr,
                        jnp.int32(1), jnp.int32(0))
                acc = lax.fori_loop(0, nv, cbody, jnp.zeros((_L,), jnp.int32))
                return jnp.sum(acc)

            def bit(b, kstar):
                cand = kstar + (jnp.uint32(1) << (jnp.uint32(31) - b.astype(jnp.uint32)))
                cand = jnp.where(cand < kstar, kstar, cand)
                c1 = count_ge(cand)
                return jnp.where(c1 >= k, cand, kstar)

            kstar = lax.fori_loop(0, 32, bit, jnp.uint32(0))
            c_gt = count_ge(kstar + jnp.uint32(1))
            c_gt = jnp.where(kstar == jnp.uint32(0xFFFFFFFF), 0, c_gt)
            quota = jnp.int32(k) - c_gt

            def emit(j, carry):
                off, used = carry
                sl = pl.ds(j * _L, _L)
                kvj = kv[sl]
                m_gt = kvj > kstar
                m_eq = kvj == kstar
                eqi = jnp.where(m_eq, jnp.int32(1), jnp.int32(0))
                eqc = plsc.cumsum(eqi)
                sel = m_gt | (m_eq & ((used + eqc) <= quota))
                si = jnp.where(sel, jnp.int32(1), jnp.int32(0))
                pos = off + plsc.cumsum(si) - si
                gidx = j * _L + lax.iota(jnp.int32, _L)
                plsc.store_scatter(pidx, (pos,), gidx, mask=sel)
                plsc.store_scatter(pval, (pos,), sv[sl], mask=sel)
                return off + jnp.sum(si), used + jnp.sum(eqi)

            lax.fori_loop(0, nv, emit, (jnp.int32(0), jnp.int32(0)))
            pltpu.sync_copy(pidx.at[pl.ds(0, k)], perm_hbm)
            pltpu.sync_copy(pval.at[pl.ds(0, k)], vals_hbm)

    return kk(score_pad)


def _sc_pool_finish(perm, tab, src, dst, valid, n, k, n_next):
    E = src.shape[0]
    w = tab.shape[1]
    per_w = E // _NW
    kp = 8 * ((k + 8 * _NW - 1) // (8 * _NW))

    @functools.partial(
        pl.kernel,
        out_type=(jax.ShapeDtypeStruct((k, w), jnp.float32),
                  jax.ShapeDtypeStruct((E,), jnp.int32),
                  jax.ShapeDtypeStruct((E,), jnp.int32),
                  jax.ShapeDtypeStruct((E,), jnp.int32),
                  jax.ShapeDtypeStruct((E,), jnp.int32)),
        mesh=plsc.VectorSubcoreMesh(**_MESH),
        scratch_types=[
            pltpu.VMEM((k,), jnp.int32),
            pltpu.VMEM((n,), jnp.int32),
            pltpu.VMEM((kp,), jnp.int32),
            pltpu.VMEM((kp, w), jnp.float32),
            pltpu.VMEM((per_w,), jnp.int32),
            pltpu.VMEM((per_w,), jnp.int32),
            pltpu.VMEM((per_w,), jnp.int32),
            pltpu.VMEM((per_w,), jnp.int32),
            pltpu.VMEM((per_w,), jnp.int32),
            pltpu.VMEM((per_w,), jnp.int32),
            pltpu.VMEM((per_w,), jnp.int32),
            pltpu.SemaphoreType.DMA,
        ],
        compiler_params=_SC_PARAMS,
    )
    def kk(perm_hbm, tab_hbm, src_hbm, dst_hbm, val_hbm,
           xout_hbm, ns_hbm, nd_hbm, v_hbm, seg_hbm,
           permv, nidx, gi, rows, sv, dv, vv, nsv, ndv, vov, segv, sem):
        c = lax.axis_index("c")
        s = lax.axis_index("s")
        wid = s * _NC + c
        base = wid * per_w
        pltpu.sync_copy(perm_hbm, permv)
        pltpu.sync_copy(src_hbm.at[pl.ds(base, per_w)], sv)
        pltpu.sync_copy(dst_hbm.at[pl.ds(base, per_w)], dv)
        pltpu.sync_copy(val_hbm.at[pl.ds(base, per_w)], vv)

        gb = jnp.minimum(wid * kp, k - kp)
        pltpu.sync_copy(perm_hbm.at[pl.ds(gb, kp)], gi)
        pltpu.async_copy(tab_hbm.at[gi], rows, sem).wait()
        pltpu.sync_copy(rows, xout_hbm.at[pl.ds(gb, kp)])

        def initn(j, _):
            nidx[pl.ds(j * _L, _L)] = jnp.full((_L,), -1, jnp.int32)
            return 0

        lax.fori_loop(0, n // _L, initn, 0)

        def scat(j, _):
            pv = permv[pl.ds(j * _L, _L)]
            plsc.store_scatter(nidx, (pv,), j * _L + lax.iota(jnp.int32, _L))
            return 0

        lax.fori_loop(0, k // _L, scat, 0)

        def relab(j, _):
            sl = pl.ds(j * _L, _L)
            ns = plsc.load_gather(nidx, (sv[sl],))
            nd = plsc.load_gather(nidx, (dv[sl],))
            ok = (vv[sl] != 0) & (ns >= 0) & (nd >= 0)
            nsv[sl] = jnp.where(ok, ns, 0)
            ndv[sl] = jnp.where(ok, nd, 0)
            vov[sl] = jnp.where(ok, jnp.int32(1), jnp.int32(0))
            eid = (j * _L + lax.iota(jnp.int32, _L)) & (_DSPREAD - 1)
            segv[sl] = jnp.where(ok, nd, n_next + 1 + eid)
            return 0

        lax.fori_loop(0, per_w // _L, relab, 0)
        pltpu.sync_copy(nsv, ns_hbm.at[pl.ds(base, per_w)])
        pltpu.sync_copy(ndv, nd_hbm.at[pl.ds(base, per_w)])
        pltpu.sync_copy(vov, v_hbm.at[pl.ds(base, per_w)])
        pltpu.sync_copy(segv, seg_hbm.at[pl.ds(base, per_w)])

    return kk(perm, tab, src, dst, valid)


def _conv(x, e, src2, seg2, p, n, spread=0):
    if "src" in p:
        xs = _mm(x, p["src"]["W"], p["src"]["b"])
        xd = _mm(x, p["dst"]["W"], p["dst"]["b"])
    else:
        xs = x
        xd = x
    parts = _sc_edge_pass(xs, e, src2, seg2, _npad(n, spread))
    scale = 1.0 / np.sqrt(1.0 + 1e-5)
    w1 = p["mlp1"]["W"] * (p["bn_gamma"] * scale)[None, :]
    b1 = p["mlp1"]["b"] * p["bn_gamma"] * scale + p["bn_beta"]
    return _combine_mlp(parts, xd, w1, b1, p["mlp2"]["W"], p["mlp2"]["b"])


def _pool(h, src, dst, valid, src2, seg2, p, n, k, spread=0):
    E = src.shape[0]
    w = h.shape[1]
    parts = _sc_agg_pass(h, src2, seg2, _npad(n, spread))
    score = _scorer(parts, h, p["rel"]["W"], p["rel"]["b"], p["root"]["W"])
    ntop = 512 * ((n + 511) // 512)
    score_pad = jnp.concatenate(
        [score.reshape(-1), jnp.full((ntop - n,), -1e30, jnp.float32)])
    perm, vals = _sc_topk(score_pad, k, n)
    x_raw, ns, nd, v, seg = _sc_pool_finish(perm, h, src, dst, valid, n, k, k)
    x_new = _scale_relu(x_raw, vals)
    return x_new, ns, nd, v, seg


def kernel(x, edge_index, edge_attr, batch, params):
    n0 = x.shape[0]
    E = edge_index.shape[1]
    src, dst = edge_index[0], edge_index[1]
    valid = jnp.ones((E,), jnp.int32)
    k1 = int(np.ceil(0.2 * n0))
    k2 = int(np.ceil(0.2 * k1))
    k3 = int(np.ceil(0.2 * k2))

    def r2(a, w):
        ch = _ch(w)
        return a.reshape(E // ch, ch)

    def r3(a, w):
        ch = _ch(w)
        return a.reshape(E // ch, ch, w)

    e1 = _mm(edge_attr, params["conv1"]["edge"]["W"], params["conv1"]["edge"]["b"])
    h = _conv(x, r3(e1, 64), r2(src, 64), r2(dst, 64), params["conv1"], n0)
    h, src, dst, valid, seg = _pool(h, src, dst, valid, r2(src, 64), r2(dst, 64),
                                    params["pool1"], n0, k1)

    e2 = _mm(edge_attr, params["conv2"]["edge"]["W"], params["conv2"]["edge"]["b"])
    h = _conv(h, r3(e2, 64), r2(src, 64), r2(seg, 64), params["conv2"], k1,
              spread=_DSPREAD + 1)
    h, src, dst, valid, seg = _pool(h, src, dst, valid, r2(src, 64), r2(seg, 64),
                                    params["pool2"], k1, k2, spread=_DSPREAD + 1)

    e3 = _mm(edge_attr, params["conv3"]["edge"]["W"], params["conv3"]["edge"]["b"])
    h = _conv(h, r3(e3, 128), r2(src, 128), r2(seg, 128), params["conv3"], k2,
              spread=_DSPREAD + 1)
    h, src, dst, valid, seg = _pool(h, src, dst, valid, r2(src, 128), r2(seg, 128),
                                    params["pool3"], k2, k3, spread=_DSPREAD + 1)

    return _head(h, params, float(k3))


# speedup vs baseline: 14.6637x; 14.6637x over previous
import functools

import jax
import jax.numpy as jnp
import numpy as np
from jax import lax
from jax.experimental import pallas as pl
from jax.experimental.pallas import tpu as pltpu
from jax.experimental.pallas import tpu_sc as plsc

_NC, _NS, _L = 2, 16, 16
_NW = _NC * _NS
_MESH = dict(core_axis_name="c", subcore_axis_name="s",
             num_cores=_NC, num_subcores=_NS)
_SC_PARAMS = pltpu.CompilerParams(use_tc_tiling_on_sc=False,
                                  needs_layout_passes=False)
_CLAMP = 60.0
_DSPREAD = 4096
_GSPREAD = 256


def _npad(n, spread=0):
    return 512 * ((n + 1 + spread + 511) // 512)


def _ch(w):
    return 80 if w <= 64 else 40


def _mm_body(a_ref, b_ref, bias_ref, o_ref):
    o_ref[...] = (
        jnp.dot(a_ref[...], b_ref[...], preferred_element_type=jnp.float32)
        + bias_ref[...]
    )


def _mm(a, b, bias=None):
    m, k = a.shape
    n = b.shape[1]
    if bias is None:
        bias = jnp.zeros((1, n), jnp.float32)
    else:
        bias = bias.reshape(1, n)
    block_m = m if m * n * 4 <= 4 * 1024 * 1024 else 8000
    return pl.pallas_call(
        _mm_body,
        grid=(m // block_m,),
        in_specs=[
            pl.BlockSpec((block_m, k), lambda i: (i, 0)),
            pl.BlockSpec((k, n), lambda i: (0, 0)),
            pl.BlockSpec((1, n), lambda i: (0, 0)),
        ],
        out_specs=pl.BlockSpec((block_m, n), lambda i: (i, 0)),
        out_shape=jax.ShapeDtypeStruct((m, n), jnp.float32),
    )(a, b, bias)


def _combine_mlp_body(p_ref, xd_ref, w1_ref, b1_ref, w2_ref, b2_ref, o_ref):
    w = xd_ref.shape[1]
    den = p_ref[0, :, :w] + p_ref[1, :, :w]
    num = p_ref[0, :, w:] + p_ref[1, :, w:]
    out = jnp.where(den > 0.0, num / jnp.where(den > 0.0, den, 1.0), 0.0)
    out = out + xd_ref[...]
    h1 = jax.nn.relu(
        jnp.dot(out, w1_ref[...], preferred_element_type=jnp.float32) + b1_ref[...]
    )
    o_ref[...] = (
        jnp.dot(h1, w2_ref[...], preferred_element_type=jnp.float32) + b2_ref[...]
    )


def _combine_mlp(parts, xd, w1, b1, w2, b2):
    n, w = xd.shape
    bn = 2000 if n >= 2000 else n
    return pl.pallas_call(
        _combine_mlp_body,
        grid=(n // bn,),
        in_specs=[
            pl.BlockSpec((2, bn, 2 * w), lambda i: (0, i, 0)),
            pl.BlockSpec((bn, w), lambda i: (i, 0)),
            pl.BlockSpec(w1.shape, lambda i: (0, 0)),
            pl.BlockSpec((1, 2 * w), lambda i: (0, 0)),
            pl.BlockSpec(w2.shape, lambda i: (0, 0)),
            pl.BlockSpec((1, w), lambda i: (0, 0)),
        ],
        out_specs=pl.BlockSpec((bn, w), lambda i: (i, 0)),
        out_shape=jax.ShapeDtypeStruct((n, w), jnp.float32),
    )(parts[:, :n, :], xd, w1, b1.reshape(1, -1), w2, b2.reshape(1, -1))


def _scorer_body(pa_ref, h_ref, wrel_ref, brel_ref, wroot_ref, o_ref):
    agg = pa_ref[0] + pa_ref[1]
    s = (
        jnp.dot(agg, wrel_ref[...], preferred_element_type=jnp.float32)
        + jnp.dot(h_ref[...], wroot_ref[...], preferred_element_type=jnp.float32)
        + brel_ref[...]
    )
    o_ref[...] = jnp.tanh(s)


def _scorer(parts, h, wrel, brel, wroot):
    n, w = h.shape
    return pl.pallas_call(
        _scorer_body,
        out_shape=jax.ShapeDtypeStruct((n, 1), jnp.float32),
    )(parts[:, :n, :], h, wrel, brel.reshape(1, 1), wroot)


def _scale_relu_body(x_ref, v_ref, o_ref):
    o_ref[...] = jax.nn.relu(x_ref[...] * v_ref[...])


def _scale_relu(x, vals):
    k, w = x.shape
    return pl.pallas_call(
        _scale_relu_body,
        out_shape=jax.ShapeDtypeStruct((k, w), jnp.float32),
    )(x, vals.reshape(k, 1))


def _head_body(h_ref, w1_ref, b1_ref, w2_ref, b2_ref, cnt_ref, o_ref):
    s = jnp.sum(h_ref[...], axis=0, keepdims=True)
    h = s / jnp.maximum(cnt_ref[0, 0], 1.0)
    h = jnp.dot(h, w1_ref[...], preferred_element_type=jnp.float32) + b1_ref[...]
    h = jnp.dot(h, w2_ref[...], preferred_element_type=jnp.float32) + b2_ref[...]
    o_ref[...] = h - jax.scipy.special.logsumexp(h, axis=-1, keepdims=True)


def _head(h, p, cnt):
    return pl.pallas_call(
        _head_body,
        out_shape=jax.ShapeDtypeStruct((1, 10), jnp.float32),
    )(h, p["dense1"]["W"], p["dense1"]["b"][None, :],
      p["dense2"]["W"], p["dense2"]["b"][None, :],
      jnp.full((1, 1), cnt, jnp.float32))


def _zero_acc(acc, zbuf, s, rows_s, width):
    zb = zbuf.shape[0]
    for r in range(zb):
        for q in range(width // _L):
            zbuf[r, pl.ds(q * _L, _L)] = jnp.zeros((_L,), jnp.float32)

    def zrow(i, _):
        pltpu.sync_copy(zbuf, acc.at[pl.ds(s * rows_s + i * zb, zb)])
        return 0

    lax.fori_loop(0, rows_s // zb, zrow, 0)


def _sc_edge_pass(tab, e3, src2, seg2, npad):
    nch, ch, w = e3.shape
    nch_w = nch // _NW
    rows_s = npad // _NS

    @functools.partial(
        pl.kernel,
        out_type=jax.ShapeDtypeStruct((_NC, npad, 2 * w), jnp.float32),
        mesh=plsc.VectorSubcoreMesh(**_MESH),
        scratch_types=[
            pltpu.VMEM((nch_w, ch), jnp.int32),
            pltpu.VMEM((nch_w, ch), jnp.int32),
            pltpu.VMEM((ch, w), jnp.float32),
            pltpu.VMEM((ch, w), jnp.float32),
            pltpu.VMEM((ch, 2 * w), jnp.float32),
            pltpu.VMEM((8, 2 * w), jnp.float32),
            pltpu.VMEM_SHARED((npad, 2 * w), jnp.float32),
            pltpu.SemaphoreType.DMA,
        ],
        compiler_params=_SC_PARAMS,
    )
    def k(tab_hbm, e_hbm, src_hbm, seg_hbm, out_hbm,
          srcv, segv, xsb, eb, stage, zbuf, acc, sem):
        c = lax.axis_index("c")
        s = lax.axis_index("s")
        wid = s * _NC + c
        _zero_acc(acc, zbuf, s, rows_s, 2 * w)
        pltpu.sync_copy(src_hbm.at[pl.ds(wid * nch_w, nch_w)], srcv)
        pltpu.sync_copy(seg_hbm.at[pl.ds(wid * nch_w, nch_w)], segv)
        plsc.subcore_barrier()

        def chunk(i, _):
            cp = pltpu.async_copy(tab_hbm.at[srcv.at[i]], xsb, sem)
            pltpu.sync_copy(e_hbm.at[wid * nch_w + i], eb)
            cp.wait()
            for r in range(ch):
                for q in range(w // _L):
                    sl = pl.ds(q * _L, _L)
                    m = jnp.maximum(xsb[r, sl] + eb[r, sl], 0.0) + 1e-7
                    ex = jnp.exp(m - _CLAMP)
                    stage[r, sl] = ex
                    stage[r, pl.ds(w + q * _L, _L)] = ex * m
            pltpu.sync_copy(stage, acc.at[segv.at[i]], add=True)
            return 0

        lax.fori_loop(0, nch_w, chunk, 0)
        plsc.subcore_barrier()
        pltpu.sync_copy(acc.at[pl.ds(s * rows_s, rows_s)],
                        out_hbm.at[c, pl.ds(s * rows_s, rows_s)])

    return k(tab, e3, src2, seg2)


def _sc_agg_pass(tab, src2, seg2, npad):
    nch, ch = src2.shape
    w = tab.shape[1]
    nch_w = nch // _NW
    rows_s = npad // _NS

    @functools.partial(
        pl.kernel,
        out_type=jax.ShapeDtypeStruct((_NC, npad, w), jnp.float32),
        mesh=plsc.VectorSubcoreMesh(**_MESH),
        scratch_types=[
            pltpu.VMEM((nch_w, ch), jnp.int32),
            pltpu.VMEM((nch_w, ch), jnp.int32),
            pltpu.VMEM((ch, w), jnp.float32),
            pltpu.VMEM((8, w), jnp.float32),
            pltpu.VMEM_SHARED((npad, w), jnp.float32),
            pltpu.SemaphoreType.DMA,
        ],
        compiler_params=_SC_PARAMS,
    )
    def k(tab_hbm, src_hbm, seg_hbm, out_hbm, srcv, segv, buf, zbuf, acc, sem):
        c = lax.axis_index("c")
        s = lax.axis_index("s")
        wid = s * _NC + c
        _zero_acc(acc, zbuf, s, rows_s, w)
        pltpu.sync_copy(src_hbm.at[pl.ds(wid * nch_w, nch_w)], srcv)
        pltpu.sync_copy(seg_hbm.at[pl.ds(wid * nch_w, nch_w)], segv)
        plsc.subcore_barrier()

        def chunk(i, _):
            pltpu.async_copy(tab_hbm.at[srcv.at[i]], buf, sem).wait()
            pltpu.sync_copy(buf, acc.at[segv.at[i]], add=True)
            return 0

        lax.fori_loop(0, nch_w, chunk, 0)
        plsc.subcore_barrier()
        pltpu.sync_copy(acc.at[pl.ds(s * rows_s, rows_s)],
                        out_hbm.at[c, pl.ds(s * rows_s, rows_s)])

    return k(tab, src2, seg2)


def _sc_topk(score_pad, k, n):
    ntop = score_pad.shape[0]
    nv = ntop // _L

    @functools.partial(
        pl.kernel,
        out_type=(jax.ShapeDtypeStruct((k,), jnp.int32),
                  jax.ShapeDtypeStruct((k,), jnp.float32)),
        mesh=plsc.VectorSubcoreMesh(**_MESH),
        scratch_types=[
            pltpu.VMEM((ntop,), jnp.float32),
            pltpu.VMEM((ntop,), jnp.uint32),
            pltpu.VMEM((k + _L,), jnp.int32),
            pltpu.VMEM((k + _L,), jnp.float32),
            pltpu.SemaphoreType.DMA,
        ],
        compiler_params=_SC_PARAMS,
    )
    def kk(sc_hbm, perm_hbm, vals_hbm, sv, kv, pidx, pval, sem):
        c = lax.axis_index("c")
        s = lax.axis_index("s")
        wid = s * _NC + c

        @pl.when(wid == 0)
        def _():
            pltpu.sync_copy(sc_hbm, sv)

            def mkkey(j, _):
                sl = pl.ds(j * _L, _L)
                u = plsc.bitcast(sv[sl], jnp.uint32)
                sr = lax.shift_right_logical(u, jnp.uint32(31))
                msk = sr * jnp.uint32(0x7FFFFFFF) + jnp.uint32(0x80000000)
                kv[sl] = u ^ msk
                return 0

            lax.fori_loop(0, nv, mkkey, 0)

            def count_ge(thr):
                def cbody(j, acc):
                    return acc + jnp.where(
                        kv[pl.ds(j * _L, _L)] >= thr,
                        jnp.int32(1), jnp.int32(0))
                acc = lax.fori_loop(0, nv, cbody, jnp.zeros((_L,), jnp.int32))
                return jnp.sum(acc)

            def bit(b, kstar):
                cand = kstar + (jnp.uint32(1) << (jnp.uint32(31) - b.astype(jnp.uint32)))
                cand = jnp.where(cand < kstar, kstar, cand)
                c1 = count_ge(cand)
                return jnp.where(c1 >= k, cand, kstar)

            kstar = lax.fori_loop(0, 32, bit, jnp.uint32(0))
            c_gt = count_ge(kstar + jnp.uint32(1))
            c_gt = jnp.where(kstar == jnp.uint32(0xFFFFFFFF), 0, c_gt)
            quota = jnp.int32(k) - c_gt

            def emit(j, carry):
                off, used = carry
                sl = pl.ds(j * _L, _L)
                kvj = kv[sl]
                m_gt = kvj > kstar
                m_eq = kvj == kstar
                eqi = jnp.where(m_eq, jnp.int32(1), jnp.int32(0))
                eqc = plsc.cumsum(eqi)
                sel = m_gt | (m_eq & ((used + eqc) <= quota))
                si = jnp.where(sel, jnp.int32(1), jnp.int32(0))
                pos = off + plsc.cumsum(si) - si
                gidx = j * _L + lax.iota(jnp.int32, _L)
                plsc.store_scatter(pidx, (pos,), gidx, mask=sel)
                plsc.store_scatter(pval, (pos,), sv[sl], mask=sel)
                return off + jnp.sum(si), used + jnp.sum(eqi)

            lax.fori_loop(0, nv, emit, (jnp.int32(0), jnp.int32(0)))
            pltpu.sync_copy(pidx.at[pl.ds(0, k)], perm_hbm)
            pltpu.sync_copy(pval.at[pl.ds(0, k)], vals_hbm)

    return kk(score_pad)


def _sc_pool_finish(perm, tab, src, dst, valid, n, k, n_next):
    E = src.shape[0]
    w = tab.shape[1]
    per_w = E // _NW
    kp = 8 * ((k + 8 * _NW - 1) // (8 * _NW))

    @functools.partial(
        pl.kernel,
        out_type=(jax.ShapeDtypeStruct((k, w), jnp.float32),
                  jax.ShapeDtypeStruct((E,), jnp.int32),
                  jax.ShapeDtypeStruct((E,), jnp.int32),
                  jax.ShapeDtypeStruct((E,), jnp.int32),
                  jax.ShapeDtypeStruct((E,), jnp.int32)),
        mesh=plsc.VectorSubcoreMesh(**_MESH),
        scratch_types=[
            pltpu.VMEM((k,), jnp.int32),
            pltpu.VMEM((n,), jnp.int32),
            pltpu.VMEM((kp,), jnp.int32),
            pltpu.VMEM((kp, w), jnp.float32),
            pltpu.VMEM((per_w,), jnp.int32),
            pltpu.VMEM((per_w,), jnp.int32),
            pltpu.VMEM((per_w,), jnp.int32),
            pltpu.VMEM((per_w,), jnp.int32),
            pltpu.VMEM((per_w,), jnp.int32),
            pltpu.VMEM((per_w,), jnp.int32),
            pltpu.VMEM((per_w,), jnp.int32),
            pltpu.SemaphoreType.DMA,
        ],
        compiler_params=_SC_PARAMS,
    )
    def kk(perm_hbm, tab_hbm, src_hbm, dst_hbm, val_hbm,
           xout_hbm, ns_hbm, nd_hbm, v_hbm, seg_hbm,
           permv, nidx, gi, rows, sv, dv, vv, nsv, ndv, vov, segv, sem):
        c = lax.axis_index("c")
        s = lax.axis_index("s")
        wid = s * _NC + c
        base = wid * per_w
        pltpu.sync_copy(perm_hbm, permv)
        pltpu.sync_copy(src_hbm.at[pl.ds(base, per_w)], sv)
        pltpu.sync_copy(dst_hbm.at[pl.ds(base, per_w)], dv)
        pltpu.sync_copy(val_hbm.at[pl.ds(base, per_w)], vv)

        gb = jnp.minimum(wid * kp, k - kp)
        pltpu.sync_copy(perm_hbm.at[pl.ds(gb, kp)], gi)
        pltpu.async_copy(tab_hbm.at[gi], rows, sem).wait()
        pltpu.sync_copy(rows, xout_hbm.at[pl.ds(gb, kp)])

        def initn(j, _):
            nidx[pl.ds(j * _L, _L)] = jnp.full((_L,), -1, jnp.int32)
            return 0

        lax.fori_loop(0, n // _L, initn, 0)

        def scat(j, _):
            pv = permv[pl.ds(j * _L, _L)]
            plsc.store_scatter(nidx, (pv,), j * _L + lax.iota(jnp.int32, _L))
            return 0

        lax.fori_loop(0, k // _L, scat, 0)

        def relab(j, _):
            sl = pl.ds(j * _L, _L)
            ns = plsc.load_gather(nidx, (sv[sl],))
            nd = plsc.load_gather(nidx, (dv[sl],))
            ok = (vv[sl] != 0) & (ns >= 0) & (nd >= 0)
            eid0 = (j * _L + lax.iota(jnp.int32, _L)) & (_GSPREAD - 1)
            nsv[sl] = jnp.where(ok, ns, eid0)
            ndv[sl] = jnp.where(ok, nd, 0)
            vov[sl] = jnp.where(ok, jnp.int32(1), jnp.int32(0))
            eid = (j * _L + lax.iota(jnp.int32, _L)) & (_DSPREAD - 1)
            segv[sl] = jnp.where(ok, nd, n_next + 1 + eid)
            return 0

        lax.fori_loop(0, per_w // _L, relab, 0)
        pltpu.sync_copy(nsv, ns_hbm.at[pl.ds(base, per_w)])
        pltpu.sync_copy(ndv, nd_hbm.at[pl.ds(base, per_w)])
        pltpu.sync_copy(vov, v_hbm.at[pl.ds(base, per_w)])
        pltpu.sync_copy(segv, seg_hbm.at[pl.ds(base, per_w)])

    return kk(perm, tab, src, dst, valid)


def _conv(x, e, src2, seg2, p, n, spread=0):
    if "src" in p:
        xs = _mm(x, p["src"]["W"], p["src"]["b"])
        xd = _mm(x, p["dst"]["W"], p["dst"]["b"])
    else:
        xs = x
        xd = x
    parts = _sc_edge_pass(xs, e, src2, seg2, _npad(n, spread))
    scale = 1.0 / np.sqrt(1.0 + 1e-5)
    w1 = p["mlp1"]["W"] * (p["bn_gamma"] * scale)[None, :]
    b1 = p["mlp1"]["b"] * p["bn_gamma"] * scale + p["bn_beta"]
    return _combine_mlp(parts, xd, w1, b1, p["mlp2"]["W"], p["mlp2"]["b"])


def _pool(h, src, dst, valid, src2, seg2, p, n, k, spread=0):
    E = src.shape[0]
    w = h.shape[1]
    parts = _sc_agg_pass(h, src2, seg2, _npad(n, spread))
    score = _scorer(parts, h, p["rel"]["W"], p["rel"]["b"], p["root"]["W"])
    ntop = 512 * ((n + 511) // 512)
    score_pad = jnp.concatenate(
        [score.reshape(-1), jnp.full((ntop - n,), -1e30, jnp.float32)])
    perm, vals = _sc_topk(score_pad, k, n)
    x_raw, ns, nd, v, seg = _sc_pool_finish(perm, h, src, dst, valid, n, k, k)
    x_new = _scale_relu(x_raw, vals)
    return x_new, ns, nd, v, seg


def kernel(x, edge_index, edge_attr, batch, params):
    n0 = x.shape[0]
    E = edge_index.shape[1]
    src, dst = edge_index[0], edge_index[1]
    valid = jnp.ones((E,), jnp.int32)
    k1 = int(np.ceil(0.2 * n0))
    k2 = int(np.ceil(0.2 * k1))
    k3 = int(np.ceil(0.2 * k2))

    def r2(a, w):
        ch = _ch(w)
        return a.reshape(E // ch, ch)

    def r3(a, w):
        ch = _ch(w)
        return a.reshape(E // ch, ch, w)

    e1 = _mm(edge_attr, params["conv1"]["edge"]["W"], params["conv1"]["edge"]["b"])
    h = _conv(x, r3(e1, 64), r2(src, 64), r2(dst, 64), params["conv1"], n0)
    h, src, dst, valid, seg = _pool(h, src, dst, valid, r2(src, 64), r2(dst, 64),
                                    params["pool1"], n0, k1)

    e2 = _mm(edge_attr, params["conv2"]["edge"]["W"], params["conv2"]["edge"]["b"])
    h = _conv(h, r3(e2, 64), r2(src, 64), r2(seg, 64), params["conv2"], k1,
              spread=_DSPREAD + 1)
    h, src, dst, valid, seg = _pool(h, src, dst, valid, r2(src, 64), r2(seg, 64),
                                    params["pool2"], k1, k2, spread=_DSPREAD + 1)

    e3 = _mm(edge_attr, params["conv3"]["edge"]["W"], params["conv3"]["edge"]["b"])
    h = _conv(h, r3(e3, 128), r2(src, 128), r2(seg, 128), params["conv3"], k2,
              spread=_DSPREAD + 1)
    h, src, dst, valid, seg = _pool(h, src, dst, valid, r2(src, 128), r2(seg, 128),
                                    params["pool3"], k2, k3, spread=_DSPREAD + 1)

    return _head(h, params, float(k3))


# speedup vs baseline: 22.6686x; 1.5459x over previous
import functools

import jax
import jax.numpy as jnp
import numpy as np
from jax import lax
from jax.experimental import pallas as pl
from jax.experimental.pallas import tpu as pltpu
from jax.experimental.pallas import tpu_sc as plsc

_NC, _NS, _L = 2, 16, 16
_NW = _NC * _NS
_MESH = dict(core_axis_name="c", subcore_axis_name="s",
             num_cores=_NC, num_subcores=_NS)
_SC_PARAMS = pltpu.CompilerParams(use_tc_tiling_on_sc=False,
                                  needs_layout_passes=False)
_CLAMP = 60.0
_DSPREAD = 4096
_GSPREAD = 256


def _npad(n, spread=0):
    return 512 * ((n + 1 + spread + 511) // 512)


def _ch(w):
    return 80 if w <= 64 else 40


def _mm_body(a_ref, b_ref, bias_ref, o_ref):
    o_ref[...] = (
        jnp.dot(a_ref[...], b_ref[...], preferred_element_type=jnp.float32)
        + bias_ref[...]
    )


def _mm(a, b, bias=None):
    m, k = a.shape
    n = b.shape[1]
    if bias is None:
        bias = jnp.zeros((1, n), jnp.float32)
    else:
        bias = bias.reshape(1, n)
    block_m = m if m * n * 4 <= 4 * 1024 * 1024 else 8000
    return pl.pallas_call(
        _mm_body,
        grid=(m // block_m,),
        in_specs=[
            pl.BlockSpec((block_m, k), lambda i: (i, 0)),
            pl.BlockSpec((k, n), lambda i: (0, 0)),
            pl.BlockSpec((1, n), lambda i: (0, 0)),
        ],
        out_specs=pl.BlockSpec((block_m, n), lambda i: (i, 0)),
        out_shape=jax.ShapeDtypeStruct((m, n), jnp.float32),
    )(a, b, bias)


def _combine_mlp_body(p_ref, xd_ref, w1_ref, b1_ref, w2_ref, b2_ref, o_ref):
    w = xd_ref.shape[1]
    den = p_ref[0, :, :w] + p_ref[1, :, :w]
    num = p_ref[0, :, w:] + p_ref[1, :, w:]
    out = jnp.where(den > 0.0, num / jnp.where(den > 0.0, den, 1.0), 0.0)
    out = out + xd_ref[...]
    h1 = jax.nn.relu(
        jnp.dot(out, w1_ref[...], preferred_element_type=jnp.float32) + b1_ref[...]
    )
    o_ref[...] = (
        jnp.dot(h1, w2_ref[...], preferred_element_type=jnp.float32) + b2_ref[...]
    )


def _combine_mlp(parts, xd, w1, b1, w2, b2):
    n, w = xd.shape
    bn = 2000 if n >= 2000 else n
    return pl.pallas_call(
        _combine_mlp_body,
        grid=(n // bn,),
        in_specs=[
            pl.BlockSpec((2, bn, 2 * w), lambda i: (0, i, 0)),
            pl.BlockSpec((bn, w), lambda i: (i, 0)),
            pl.BlockSpec(w1.shape, lambda i: (0, 0)),
            pl.BlockSpec((1, 2 * w), lambda i: (0, 0)),
            pl.BlockSpec(w2.shape, lambda i: (0, 0)),
            pl.BlockSpec((1, w), lambda i: (0, 0)),
        ],
        out_specs=pl.BlockSpec((bn, w), lambda i: (i, 0)),
        out_shape=jax.ShapeDtypeStruct((n, w), jnp.float32),
    )(parts[:, :n, :], xd, w1, b1.reshape(1, -1), w2, b2.reshape(1, -1))


def _scorer_body(pa_ref, h_ref, wrel_ref, brel_ref, wroot_ref, o_ref):
    agg = pa_ref[0] + pa_ref[1]
    s = (
        jnp.dot(agg, wrel_ref[...], preferred_element_type=jnp.float32)
        + jnp.dot(h_ref[...], wroot_ref[...], preferred_element_type=jnp.float32)
        + brel_ref[...]
    )
    o_ref[...] = s


def _scorer(parts, h, wrel, brel, wroot):
    n, w = h.shape
    return pl.pallas_call(
        _scorer_body,
        out_shape=jax.ShapeDtypeStruct((n, 1), jnp.float32),
    )(parts[:, :n, :], h, wrel, brel.reshape(1, 1), wroot)


def _scale_relu_body(x_ref, v_ref, o_ref):
    o_ref[...] = jax.nn.relu(x_ref[...] * v_ref[...])


def _scale_relu(x, vals):
    k, w = x.shape
    return pl.pallas_call(
        _scale_relu_body,
        out_shape=jax.ShapeDtypeStruct((k, w), jnp.float32),
    )(x, vals.reshape(k, 1))


def _head_body(h_ref, w1_ref, b1_ref, w2_ref, b2_ref, cnt_ref, o_ref):
    s = jnp.sum(h_ref[...], axis=0, keepdims=True)
    h = s / jnp.maximum(cnt_ref[0, 0], 1.0)
    h = jnp.dot(h, w1_ref[...], preferred_element_type=jnp.float32) + b1_ref[...]
    h = jnp.dot(h, w2_ref[...], preferred_element_type=jnp.float32) + b2_ref[...]
    o_ref[...] = h - jax.scipy.special.logsumexp(h, axis=-1, keepdims=True)


def _head(h, p, cnt):
    return pl.pallas_call(
        _head_body,
        out_shape=jax.ShapeDtypeStruct((1, 10), jnp.float32),
    )(h, p["dense1"]["W"], p["dense1"]["b"][None, :],
      p["dense2"]["W"], p["dense2"]["b"][None, :],
      jnp.full((1, 1), cnt, jnp.float32))


def _zero_acc(acc, zbuf, s, rows_s, width):
    zb = zbuf.shape[0]
    for r in range(zb):
        for q in range(width // _L):
            zbuf[r, pl.ds(q * _L, _L)] = jnp.zeros((_L,), jnp.float32)

    def zrow(i, _):
        pltpu.sync_copy(zbuf, acc.at[pl.ds(s * rows_s + i * zb, zb)])
        return 0

    lax.fori_loop(0, rows_s // zb, zrow, 0)


def _sc_edge_pass(tab, e3, src2, seg2, npad, mtab=None):
    nch, ch, w = e3.shape
    nch_w = nch // _NW
    rows_s = npad // _NS
    pb = mtab is not None
    ow = 2 * w if pb else w
    ins = [tab, e3, src2, seg2] + ([mtab] if pb else [])

    @functools.partial(
        pl.kernel,
        out_type=jax.ShapeDtypeStruct((_NC, npad, ow), jnp.float32),
        mesh=plsc.VectorSubcoreMesh(**_MESH),
        scratch_types=[
            pltpu.VMEM((nch_w, ch), jnp.int32),
            pltpu.VMEM((nch_w, ch), jnp.int32),
            pltpu.VMEM((ch, w), jnp.float32),
            pltpu.VMEM((ch, w), jnp.float32),
            pltpu.VMEM((ch, w), jnp.float32),
            pltpu.VMEM((ch, ow), jnp.float32),
            pltpu.VMEM((8, ow), jnp.float32),
            pltpu.VMEM_SHARED((npad, ow), jnp.float32),
            pltpu.SemaphoreType.DMA,
            pltpu.SemaphoreType.DMA,
        ],
        compiler_params=_SC_PARAMS,
    )
    def k(*args):
        if pb:
            (tab_hbm, e_hbm, src_hbm, seg_hbm, m_hbm, out_hbm,
             srcv, segv, xsb, eb, mb, stage, zbuf, acc, sem, sem2) = args
        else:
            (tab_hbm, e_hbm, src_hbm, seg_hbm, out_hbm,
             srcv, segv, xsb, eb, mb, stage, zbuf, acc, sem, sem2) = args
        c = lax.axis_index("c")
        s = lax.axis_index("s")
        wid = s * _NC + c
        _zero_acc(acc, zbuf, s, rows_s, ow)
        pltpu.sync_copy(src_hbm.at[pl.ds(wid * nch_w, nch_w)], srcv)
        pltpu.sync_copy(seg_hbm.at[pl.ds(wid * nch_w, nch_w)], segv)
        plsc.subcore_barrier()

        def chunk(i, _):
            cp = pltpu.async_copy(tab_hbm.at[srcv.at[i]], xsb, sem)
            if pb:
                cpm = pltpu.async_copy(m_hbm.at[segv.at[i]], mb, sem2)
            pltpu.sync_copy(e_hbm.at[wid * nch_w + i], eb)
            cp.wait()
            if pb:
                cpm.wait()
            for r in range(ch):
                for q in range(w // _L):
                    sl = pl.ds(q * _L, _L)
                    m = jnp.maximum(xsb[r, sl] + eb[r, sl], 0.0) + 1e-7
                    if pb:
                        ex = jnp.exp(m - mb[r, sl])
                        stage[r, sl] = ex
                        stage[r, pl.ds(w + q * _L, _L)] = ex * m
                    else:
                        stage[r, sl] = jnp.exp(m - _CLAMP)
            pltpu.sync_copy(stage, acc.at[segv.at[i]], add=True)
            return 0

        lax.fori_loop(0, nch_w, chunk, 0)
        plsc.subcore_barrier()
        pltpu.sync_copy(acc.at[pl.ds(s * rows_s, rows_s)],
                        out_hbm.at[c, pl.ds(s * rows_s, rows_s)])

    return k(*ins)


def _sc_agg_pass(tab, src2, seg2, npad):
    nch, ch = src2.shape
    w = tab.shape[1]
    nch_w = nch // _NW
    rows_s = npad // _NS

    @functools.partial(
        pl.kernel,
        out_type=jax.ShapeDtypeStruct((_NC, npad, w), jnp.float32),
        mesh=plsc.VectorSubcoreMesh(**_MESH),
        scratch_types=[
            pltpu.VMEM((nch_w, ch), jnp.int32),
            pltpu.VMEM((nch_w, ch), jnp.int32),
            pltpu.VMEM((ch, w), jnp.float32),
            pltpu.VMEM((8, w), jnp.float32),
            pltpu.VMEM_SHARED((npad, w), jnp.float32),
            pltpu.SemaphoreType.DMA,
        ],
        compiler_params=_SC_PARAMS,
    )
    def k(tab_hbm, src_hbm, seg_hbm, out_hbm, srcv, segv, buf, zbuf, acc, sem):
        c = lax.axis_index("c")
        s = lax.axis_index("s")
        wid = s * _NC + c
        _zero_acc(acc, zbuf, s, rows_s, w)
        pltpu.sync_copy(src_hbm.at[pl.ds(wid * nch_w, nch_w)], srcv)
        pltpu.sync_copy(seg_hbm.at[pl.ds(wid * nch_w, nch_w)], segv)
        plsc.subcore_barrier()

        def chunk(i, _):
            pltpu.async_copy(tab_hbm.at[srcv.at[i]], buf, sem).wait()
            pltpu.sync_copy(buf, acc.at[segv.at[i]], add=True)
            return 0

        lax.fori_loop(0, nch_w, chunk, 0)
        plsc.subcore_barrier()
        pltpu.sync_copy(acc.at[pl.ds(s * rows_s, rows_s)],
                        out_hbm.at[c, pl.ds(s * rows_s, rows_s)])

    return k(tab, src2, seg2)


def _sc_topk(score_pad, k, n):
    ntop = score_pad.shape[0]
    nv = ntop // _L

    @functools.partial(
        pl.kernel,
        out_type=(jax.ShapeDtypeStruct((k,), jnp.int32),
                  jax.ShapeDtypeStruct((k,), jnp.float32)),
        mesh=plsc.VectorSubcoreMesh(**_MESH),
        scratch_types=[
            pltpu.VMEM((ntop,), jnp.float32),
            pltpu.VMEM((ntop,), jnp.uint32),
            pltpu.VMEM((k + _L,), jnp.int32),
            pltpu.VMEM((k + _L,), jnp.float32),
            pltpu.SemaphoreType.DMA,
        ],
        compiler_params=_SC_PARAMS,
    )
    def kk(sc_hbm, perm_hbm, vals_hbm, sv, kv, pidx, pval, sem):
        c = lax.axis_index("c")
        s = lax.axis_index("s")
        wid = s * _NC + c

        @pl.when(wid == 0)
        def _():
            pltpu.sync_copy(sc_hbm, sv)

            def mkkey(j, _):
                sl = pl.ds(j * _L, _L)
                u = plsc.bitcast(sv[sl], jnp.uint32)
                sr = lax.shift_right_logical(u, jnp.uint32(31))
                msk = sr * jnp.uint32(0x7FFFFFFF) + jnp.uint32(0x80000000)
                kv[sl] = u ^ msk
                return 0

            lax.fori_loop(0, nv, mkkey, 0)

            def count_ge(thr):
                def cbody(j, acc):
                    return acc + jnp.where(
                        kv[pl.ds(j * _L, _L)] >= thr,
                        jnp.int32(1), jnp.int32(0))
                acc = lax.fori_loop(0, nv, cbody, jnp.zeros((_L,), jnp.int32))
                return jnp.sum(acc)

            def bit(b, kstar):
                cand = kstar + (jnp.uint32(1) << (jnp.uint32(31) - b.astype(jnp.uint32)))
                cand = jnp.where(cand < kstar, kstar, cand)
                c1 = count_ge(cand)
                return jnp.where(c1 >= k, cand, kstar)

            kstar = lax.fori_loop(0, 32, bit, jnp.uint32(0))
            c_gt = count_ge(kstar + jnp.uint32(1))
            c_gt = jnp.where(kstar == jnp.uint32(0xFFFFFFFF), 0, c_gt)
            quota = jnp.int32(k) - c_gt

            def emit(j, carry):
                off, used = carry
                sl = pl.ds(j * _L, _L)
                kvj = kv[sl]
                m_gt = kvj > kstar
                m_eq = kvj == kstar
                eqi = jnp.where(m_eq, jnp.int32(1), jnp.int32(0))
                eqc = plsc.cumsum(eqi)
                sel = m_gt | (m_eq & ((used + eqc) <= quota))
                si = jnp.where(sel, jnp.int32(1), jnp.int32(0))
                pos = off + plsc.cumsum(si) - si
                gidx = j * _L + lax.iota(jnp.int32, _L)
                plsc.store_scatter(pidx, (pos,), gidx, mask=sel)
                plsc.store_scatter(pval, (pos,), sv[sl], mask=sel)
                return off + jnp.sum(si), used + jnp.sum(eqi)

            lax.fori_loop(0, nv, emit, (jnp.int32(0), jnp.int32(0)))
            pltpu.sync_copy(pidx.at[pl.ds(0, k)], perm_hbm)
            pltpu.sync_copy(pval.at[pl.ds(0, k)], vals_hbm)

    return kk(score_pad)


_CHPAD = 80


def _sc_pool_finish_c(perm, tab, a_src, a_dst, a_eid, counts_in, n, k):
    first = a_eid is None
    E = a_src.shape[0]
    w = tab.shape[1]
    per_w = E // _NW
    kp = 8 * ((k + 8 * _NW - 1) // (8 * _NW))
    n_next = k

    ins = [perm, tab, a_src, a_dst]
    if not first:
        ins += [a_eid, counts_in]

    @functools.partial(
        pl.kernel,
        out_type=(jax.ShapeDtypeStruct((k, w), jnp.float32),
                  jax.ShapeDtypeStruct((E,), jnp.int32),
                  jax.ShapeDtypeStruct((E,), jnp.int32),
                  jax.ShapeDtypeStruct((E,), jnp.int32),
                  jax.ShapeDtypeStruct((_NW, _L), jnp.int32)),
        mesh=plsc.VectorSubcoreMesh(**_MESH),
        scratch_types=[
            pltpu.VMEM((k,), jnp.int32),
            pltpu.VMEM((n,), jnp.int32),
            pltpu.VMEM((kp,), jnp.int32),
            pltpu.VMEM((kp, w), jnp.float32),
            pltpu.VMEM((per_w,), jnp.int32),
            pltpu.VMEM((per_w,), jnp.int32),
            pltpu.VMEM((per_w,), jnp.int32),
            pltpu.VMEM((per_w,), jnp.int32),
            pltpu.VMEM((per_w,), jnp.int32),
            pltpu.VMEM((per_w,), jnp.int32),
            pltpu.VMEM((_NW, _L), jnp.int32),
            pltpu.SemaphoreType.DMA,
        ],
        compiler_params=_SC_PARAMS,
    )
    def kk(*args):
        (perm_hbm, tab_hbm, src_hbm, dst_hbm) = args[:4]
        if first:
            (xout_hbm, cns_hbm, cseg_hbm, ceid_hbm, cnt_hbm,
             permv, nidx, gi, rows, sv, dv, ev, nsv, segst, eidst,
             cntv, sem) = args[4:]
        else:
            (eid_hbm, cntin_hbm,
             xout_hbm, cns_hbm, cseg_hbm, ceid_hbm, cnt_hbm,
             permv, nidx, gi, rows, sv, dv, ev, nsv, segst, eidst,
             cntv, sem) = args[4:]
        c = lax.axis_index("c")
        s = lax.axis_index("s")
        wid = s * _NC + c
        base = wid * per_w
        pltpu.sync_copy(perm_hbm, permv)
        pltpu.sync_copy(src_hbm.at[pl.ds(base, per_w)], sv)
        pltpu.sync_copy(dst_hbm.at[pl.ds(base, per_w)], dv)
        if first:
            raw = jnp.int32(per_w)
        else:
            pltpu.sync_copy(eid_hbm.at[pl.ds(base, per_w)], ev)
            pltpu.sync_copy(cntin_hbm, cntv)
            raw = cntv[wid, pl.ds(0, _L)][1]

        gb = jnp.minimum(wid * kp, k - kp)
        pltpu.sync_copy(perm_hbm.at[pl.ds(gb, kp)], gi)
        pltpu.async_copy(tab_hbm.at[gi], rows, sem).wait()
        pltpu.sync_copy(rows, xout_hbm.at[pl.ds(gb, kp)])

        def initn(j, _):
            nidx[pl.ds(j * _L, _L)] = jnp.full((_L,), -1, jnp.int32)
            return 0

        lax.fori_loop(0, n // _L, initn, 0)

        def scat(j, _):
            pv = permv[pl.ds(j * _L, _L)]
            plsc.store_scatter(nidx, (pv,), j * _L + lax.iota(jnp.int32, _L))
            return 0

        lax.fori_loop(0, k // _L, scat, 0)

        def relab(j, off):
            sl = pl.ds(j * _L, _L)
            lane = j * _L + lax.iota(jnp.int32, _L)
            svl = jnp.minimum(sv[sl], n - 1)
            dvl = jnp.minimum(dv[sl], n - 1)
            ns = plsc.load_gather(nidx, (svl,))
            nd = plsc.load_gather(nidx, (dvl,))
            ok = (lane < raw) & (ns >= 0) & (nd >= 0)
            si = jnp.where(ok, jnp.int32(1), jnp.int32(0))
            pos = off + plsc.cumsum(si) - si
            eidv = (lane + base) if first else ev[sl]
            plsc.store_scatter(nsv, (pos,), ns, mask=ok)
            plsc.store_scatter(segst, (pos,), nd, mask=ok)
            plsc.store_scatter(eidst, (pos,), eidv, mask=ok)
            return off + jnp.sum(si)

        nvr = per_w // _L if first else (raw + (_L - 1)) // _L
        cnt = lax.fori_loop(0, nvr, relab, jnp.int32(0))

        cntp = ((cnt + (_CHPAD - 1)) // _CHPAD) * _CHPAD
        for p in range(_CHPAD // _L):
            pos = cnt + p * _L + lax.iota(jnp.int32, _L)
            mk = pos < cntp
            plsc.store_scatter(nsv, (pos,), pos & (_GSPREAD - 1), mask=mk)
            plsc.store_scatter(segst, (pos,),
                               n_next + 1 + (pos & (_DSPREAD - 1)), mask=mk)
            plsc.store_scatter(eidst, (pos,), pos & (_GSPREAD - 1), mask=mk)

        pltpu.sync_copy(nsv, cns_hbm.at[pl.ds(base, per_w)])
        pltpu.sync_copy(segst, cseg_hbm.at[pl.ds(base, per_w)])
        pltpu.sync_copy(eidst, ceid_hbm.at[pl.ds(base, per_w)])
        lanev = lax.iota(jnp.int32, _L)
        cvec = jnp.where(lanev == 0, cntp, jnp.where(lanev == 1, cnt, 0))
        cntv[wid, pl.ds(0, _L)] = cvec
        pltpu.sync_copy(cntv.at[wid], cnt_hbm.at[wid])

    return kk(*ins)


def _sc_edge_c(tab, attr, wet, be, cns2, cseg2, ceid2, counts, npad,
               mtab=None):
    nch, ch = cns2.shape
    w = tab.shape[1]
    ed = attr.shape[1]
    nch_w = nch // _NW
    rows_s = npad // _NS
    pb = mtab is not None
    ow = 2 * w if pb else w
    ins = [tab, attr, wet, be, cns2, cseg2, ceid2, counts] + ([mtab] if pb else [])

    @functools.partial(
        pl.kernel,
        out_type=jax.ShapeDtypeStruct((_NC, npad, ow), jnp.float32),
        mesh=plsc.VectorSubcoreMesh(**_MESH),
        scratch_types=[
            pltpu.VMEM((nch_w, ch), jnp.int32),
            pltpu.VMEM((nch_w, ch), jnp.int32),
            pltpu.VMEM((nch_w, ch), jnp.int32),
            pltpu.VMEM((ch, w), jnp.float32),
            pltpu.VMEM((ch, ed), jnp.float32),
            pltpu.VMEM((ch, w), jnp.float32),
            pltpu.VMEM((ed, w), jnp.float32),
            pltpu.VMEM((w,), jnp.float32),
            pltpu.VMEM((ch, ow), jnp.float32),
            pltpu.VMEM((8, ow), jnp.float32),
            pltpu.VMEM((_NW, _L), jnp.int32),
            pltpu.VMEM_SHARED((npad, ow), jnp.float32),
            pltpu.SemaphoreType.DMA,
            pltpu.SemaphoreType.DMA,
            pltpu.SemaphoreType.DMA,
        ],
        compiler_params=_SC_PARAMS,
    )
    def k(*args):
        if pb:
            (tab_hbm, attr_hbm, we_hbm, be_hbm, cns_hbm, cseg_hbm, ceid_hbm,
             cnt_hbm, m_hbm, out_hbm,
             cnsv, csegv, ceidv, xsb, atb, mb, wev, bev, stage, zbuf, cntv,
             acc, sem1, sem2, sem3) = args
        else:
            (tab_hbm, attr_hbm, we_hbm, be_hbm, cns_hbm, cseg_hbm, ceid_hbm,
             cnt_hbm, out_hbm,
             cnsv, csegv, ceidv, xsb, atb, mb, wev, bev, stage, zbuf, cntv,
             acc, sem1, sem2, sem3) = args
        c = lax.axis_index("c")
        s = lax.axis_index("s")
        wid = s * _NC + c
        _zero_acc(acc, zbuf, s, rows_s, ow)
        pltpu.sync_copy(we_hbm, wev)
        pltpu.sync_copy(be_hbm, bev)
        pltpu.sync_copy(cnt_hbm, cntv)
        pltpu.sync_copy(cns_hbm.at[pl.ds(wid * nch_w, nch_w)], cnsv)
        pltpu.sync_copy(cseg_hbm.at[pl.ds(wid * nch_w, nch_w)], csegv)
        pltpu.sync_copy(ceid_hbm.at[pl.ds(wid * nch_w, nch_w)], ceidv)
        plsc.subcore_barrier()
        nci = cntv[wid, pl.ds(0, _L)][0] // ch

        def chunk(i, _):
            cp1 = pltpu.async_copy(tab_hbm.at[cnsv.at[i]], xsb, sem1)
            cp2 = pltpu.async_copy(attr_hbm.at[ceidv.at[i]], atb, sem2)
            if pb:
                cp3 = pltpu.async_copy(m_hbm.at[csegv.at[i]], mb, sem3)
            cp1.wait()
            cp2.wait()
            if pb:
                cp3.wait()

            def row(r, _2):
                evs = [bev[pl.ds(q * _L, _L)] for q in range(w // _L)]
                av = atb[r, pl.ds(0, ed)]
                for t in range(ed):
                    a = av[t]
                    for q in range(w // _L):
                        evs[q] = evs[q] + a * wev[t, pl.ds(q * _L, _L)]
                for q in range(w // _L):
                    sl = pl.ds(q * _L, _L)
                    m = jnp.maximum(xsb[r, sl] + evs[q], 0.0) + 1e-7
                    if pb:
                        ex = jnp.exp(m - mb[r, sl])
                        stage[r, sl] = ex
                        stage[r, pl.ds(w + q * _L, _L)] = ex * m
                    else:
                        stage[r, sl] = jnp.exp(m - _CLAMP)
                return 0

            lax.fori_loop(0, ch, row, 0)
            pltpu.sync_copy(stage, acc.at[csegv.at[i]], add=True)
            return 0

        lax.fori_loop(0, nci, chunk, 0)
        plsc.subcore_barrier()
        pltpu.sync_copy(acc.at[pl.ds(s * rows_s, rows_s)],
                        out_hbm.at[c, pl.ds(s * rows_s, rows_s)])

    return k(*ins)


def _sc_agg_c(tab, cns2, cseg2, counts, npad):
    nch, ch = cns2.shape
    w = tab.shape[1]
    nch_w = nch // _NW
    rows_s = npad // _NS

    @functools.partial(
        pl.kernel,
        out_type=jax.ShapeDtypeStruct((_NC, npad, w), jnp.float32),
        mesh=plsc.VectorSubcoreMesh(**_MESH),
        scratch_types=[
            pltpu.VMEM((nch_w, ch), jnp.int32),
            pltpu.VMEM((nch_w, ch), jnp.int32),
            pltpu.VMEM((ch, w), jnp.float32),
            pltpu.VMEM((8, w), jnp.float32),
            pltpu.VMEM((_NW, _L), jnp.int32),
            pltpu.VMEM_SHARED((npad, w), jnp.float32),
            pltpu.SemaphoreType.DMA,
        ],
        compiler_params=_SC_PARAMS,
    )
    def k(tab_hbm, cns_hbm, cseg_hbm, cnt_hbm, out_hbm,
          cnsv, csegv, buf, zbuf, cntv, acc, sem):
        c = lax.axis_index("c")
        s = lax.axis_index("s")
        wid = s * _NC + c
        _zero_acc(acc, zbuf, s, rows_s, w)
        pltpu.sync_copy(cnt_hbm, cntv)
        pltpu.sync_copy(cns_hbm.at[pl.ds(wid * nch_w, nch_w)], cnsv)
        pltpu.sync_copy(cseg_hbm.at[pl.ds(wid * nch_w, nch_w)], csegv)
        plsc.subcore_barrier()
        nci = cntv[wid, pl.ds(0, _L)][0] // ch

        def chunk(i, _):
            pltpu.async_copy(tab_hbm.at[cnsv.at[i]], buf, sem).wait()
            pltpu.sync_copy(buf, acc.at[csegv.at[i]], add=True)
            return 0

        lax.fori_loop(0, nci, chunk, 0)
        plsc.subcore_barrier()
        pltpu.sync_copy(acc.at[pl.ds(s * rows_s, rows_s)],
                        out_hbm.at[c, pl.ds(s * rows_s, rows_s)])

    return k(tab, cns2, cseg2, counts)


def _sc_gather_rows(perm, tab, k):
    w = tab.shape[1]
    kp = 8 * ((k + 8 * _NW - 1) // (8 * _NW))

    @functools.partial(
        pl.kernel,
        out_type=jax.ShapeDtypeStruct((k, w), jnp.float32),
        mesh=plsc.VectorSubcoreMesh(**_MESH),
        scratch_types=[
            pltpu.VMEM((kp,), jnp.int32),
            pltpu.VMEM((kp, w), jnp.float32),
            pltpu.SemaphoreType.DMA,
        ],
        compiler_params=_SC_PARAMS,
    )
    def kk(perm_hbm, tab_hbm, xout_hbm, gi, rows, sem):
        c = lax.axis_index("c")
        s = lax.axis_index("s")
        wid = s * _NC + c
        gb = jnp.minimum(wid * kp, k - kp)
        pltpu.sync_copy(perm_hbm.at[pl.ds(gb, kp)], gi)
        pltpu.async_copy(tab_hbm.at[gi], rows, sem).wait()
        pltpu.sync_copy(rows, xout_hbm.at[pl.ds(gb, kp)])

    return kk(perm, tab)


def _logm_body(pa_ref, o_ref):
    den = pa_ref[0] + pa_ref[1]
    o_ref[...] = jnp.where(den > 0.0,
                           _CLAMP + jnp.log(jnp.where(den > 0.0, den, 1.0)),
                           0.0)


def _logm(parts):
    _, npad, w = parts.shape[0], parts.shape[1], parts.shape[2]
    return pl.pallas_call(
        _logm_body,
        out_shape=jax.ShapeDtypeStruct((npad, w), jnp.float32),
    )(parts)


def _mlp(parts, xd, p):
    scale = 1.0 / np.sqrt(1.0 + 1e-5)
    w1 = p["mlp1"]["W"] * (p["bn_gamma"] * scale)[None, :]
    b1 = p["mlp1"]["b"] * p["bn_gamma"] * scale + p["bn_beta"]
    return _combine_mlp(parts, xd, w1, b1, p["mlp2"]["W"], p["mlp2"]["b"])


def _score_topk(parts, h, p, n, k):
    score = jnp.tanh(_scorer(parts, h, p["rel"]["W"], p["rel"]["b"],
                             p["root"]["W"]))
    ntop = 512 * ((n + 511) // 512)
    score_pad = jnp.concatenate(
        [score.reshape(-1), jnp.full((ntop - n,), -1e30, jnp.float32)])
    perm, vals = _sc_topk(score_pad, k, n)
    order = jnp.argsort(-vals, stable=True)
    return perm[order], vals[order]


def kernel(x, edge_index, edge_attr, batch, params):
    n0 = x.shape[0]
    E = edge_index.shape[1]
    src, dst = edge_index[0], edge_index[1]
    k1 = int(np.ceil(0.2 * n0))
    k2 = int(np.ceil(0.2 * k1))
    k3 = int(np.ceil(0.2 * k2))

    def r2(a, w):
        ch = _ch(w)
        return a.reshape(E // ch, ch)

    p1 = params["conv1"]
    xs = _mm(x, p1["src"]["W"], p1["src"]["b"])
    xd = _mm(x, p1["dst"]["W"], p1["dst"]["b"])
    e1 = _mm(edge_attr, p1["edge"]["W"], p1["edge"]["b"])
    e1r = e1.reshape(E // 80, 80, 64)
    partsA = _sc_edge_pass(xs, e1r, r2(src, 64), r2(dst, 64), _npad(n0))
    m1 = _logm(partsA)
    parts = _sc_edge_pass(xs, e1r, r2(src, 64), r2(dst, 64), _npad(n0), m1)
    h = _mlp(parts, xd, p1)

    parts = _sc_agg_pass(h, r2(src, 64), r2(dst, 64), _npad(n0))
    perm, vals = _score_topk(parts, h, params["pool1"], n0, k1)
    x_raw, cns, cseg, ceid, cnts = _sc_pool_finish_c(
        perm, h, src, dst, None, None, n0, k1)
    h = _scale_relu(x_raw, vals)

    p2 = params["conv2"]
    np2 = _npad(k1, _DSPREAD + 1)
    partsA = _sc_edge_c(h, edge_attr, p2["edge"]["W"], p2["edge"]["b"],
                        r2(cns, 64), r2(cseg, 64), r2(ceid, 64), cnts, np2)
    m2 = _logm(partsA)
    parts = _sc_edge_c(h, edge_attr, p2["edge"]["W"], p2["edge"]["b"],
                       r2(cns, 64), r2(cseg, 64), r2(ceid, 64), cnts, np2, m2)
    h = _mlp(parts, h, p2)

    parts = _sc_agg_c(h, r2(cns, 64), r2(cseg, 64), cnts, np2)
    perm, vals = _score_topk(parts, h, params["pool2"], k1, k2)
    x_raw, cns, cseg, ceid, cnts = _sc_pool_finish_c(
        perm, h, cns, cseg, ceid, cnts, k1, k2)
    h = _scale_relu(x_raw, vals)

    p3 = params["conv3"]
    np3 = _npad(k2, _DSPREAD + 1)
    xs = _mm(h, p3["src"]["W"], p3["src"]["b"])
    xd = _mm(h, p3["dst"]["W"], p3["dst"]["b"])
    partsA = _sc_edge_c(xs, edge_attr, p3["edge"]["W"], p3["edge"]["b"],
                        r2(cns, 128), r2(cseg, 128), r2(ceid, 128), cnts, np3)
    m3 = _logm(partsA)
    parts = _sc_edge_c(xs, edge_attr, p3["edge"]["W"], p3["edge"]["b"],
                       r2(cns, 128), r2(cseg, 128), r2(ceid, 128), cnts, np3, m3)
    h = _mlp(parts, xd, p3)

    parts = _sc_agg_c(h, r2(cns, 128), r2(cseg, 128), cnts, np3)
    perm, vals = _score_topk(parts, h, params["pool3"], k2, k3)
    x_raw = _sc_gather_rows(perm, h, k3)
    h = _scale_relu(x_raw, vals)

    return _head(h, params, float(k3))
